# baseline (speedup 1.0000x reference)
import jax
import jax.numpy as jnp
from jax import lax
from jax.experimental import pallas as pl
from jax.experimental.pallas import tpu as pltpu
from jax.experimental.pallas import tpu_sc as plsc

VOCAB = 1000000
SEQ_LEN = 200
EMBED = 32
BATCH = 4096

NC, NS = 2, 16
NW = NC * NS
B_PER_W = BATCH // NW
CB = 8
NCHUNKS = B_PER_W // CB
SPLIT = 128


def _body(tok_hbm, idx_hbm, pos_hbm, out_hbm, idx_v, rows_v, pos_v, sem):
    wid = lax.axis_index("s") * NC + lax.axis_index("c")
    base_b = wid * B_PER_W

    pltpu.sync_copy(pos_hbm, pos_v)

    def chunk_body(ci, _):
        b0 = pl.multiple_of(base_b + ci * CB, CB)
        pltpu.sync_copy(idx_hbm.at[pl.ds(b0, CB)], idx_v)
        for b in range(CB):
            pltpu.async_copy(
                tok_hbm.at[idx_v.at[b, pl.ds(0, SPLIT)]],
                rows_v.at[b, pl.ds(0, SPLIT)],
                sem,
            )
            pltpu.async_copy(
                tok_hbm.at[idx_v.at[b, pl.ds(SPLIT, SEQ_LEN - SPLIT)]],
                rows_v.at[b, pl.ds(SPLIT, SEQ_LEN - SPLIT)],
                sem,
            )
        for b in range(CB):
            pltpu.make_async_copy(
                tok_hbm.at[idx_v.at[b, pl.ds(0, SPLIT)]],
                rows_v.at[b, pl.ds(0, SPLIT)],
                sem,
            ).wait()
            pltpu.make_async_copy(
                tok_hbm.at[idx_v.at[b, pl.ds(SPLIT, SEQ_LEN - SPLIT)]],
                rows_v.at[b, pl.ds(SPLIT, SEQ_LEN - SPLIT)],
                sem,
            ).wait()

        def add_body(l, _):
            p0 = pos_v[l, 0:16]
            p1 = pos_v[l, 16:32]
            for b in range(CB):
                rows_v[b, l, 0:16] = rows_v[b, l, 0:16] + p0
                rows_v[b, l, 16:32] = rows_v[b, l, 16:32] + p1
            return 0

        lax.fori_loop(0, SEQ_LEN, add_body, 0)

        pltpu.sync_copy(rows_v, out_hbm.at[pl.ds(b0, CB)])
        return 0

    lax.fori_loop(0, NCHUNKS, chunk_body, 0)


@jax.jit
def _run(tok, idx, pos):
    mesh = plsc.VectorSubcoreMesh(
        core_axis_name="c", subcore_axis_name="s", num_cores=NC, num_subcores=NS
    )
    return pl.kernel(
        _body,
        out_type=jax.ShapeDtypeStruct((BATCH, SEQ_LEN, EMBED), jnp.float32),
        mesh=mesh,
        scratch_types=[
            pltpu.VMEM((CB, SEQ_LEN), jnp.int32),
            pltpu.VMEM((CB, SEQ_LEN, EMBED), jnp.float32),
            pltpu.VMEM((SEQ_LEN, EMBED), jnp.float32),
            pltpu.SemaphoreType.DMA,
        ],
        compiler_params=pltpu.CompilerParams(use_tc_tiling_on_sc=False),
    )(tok, idx, pos)


def kernel(inputs, token_table, pos_table):
    return _run(token_table, inputs, pos_table)


# speedup vs baseline: 1.3405x; 1.3405x over previous
import jax
import jax.numpy as jnp
from jax import lax
from jax.experimental import pallas as pl
from jax.experimental.pallas import tpu as pltpu
from jax.experimental.pallas import tpu_sc as plsc

VOCAB = 1000000
SEQ_LEN = 200
EMBED = 32
PAD = 128
LROW = 128
SEQ_PAD = 2 * LROW
BATCH = 4096

NC, NS = 2, 16
NW = NC * NS
B_PER_W = BATCH // NW
CB = 8
NCHUNKS = B_PER_W // CB
SPLIT = 128

_MESH = plsc.VectorSubcoreMesh(
    core_axis_name="c", subcore_axis_name="s", num_cores=NC, num_subcores=NS
)


def _body(tok_hbm, idx_hbm, pos_hbm, out_hbm, idx_v, rows_v, pos_v, sem):
    wid = lax.axis_index("s") * NC + lax.axis_index("c")
    base_b = wid * B_PER_W

    pltpu.sync_copy(pos_hbm, pos_v)

    def chunk_body(ci, _):
        b0 = pl.multiple_of(base_b + ci * CB, CB)
        pltpu.sync_copy(idx_hbm.at[pl.ds(2 * b0, 2 * CB)], idx_v)
        for b in range(CB):
            pltpu.async_copy(
                tok_hbm.at[idx_v.at[2 * b]],
                rows_v.at[b, pl.ds(0, SPLIT)],
                sem,
            )
            pltpu.async_copy(
                tok_hbm.at[idx_v.at[2 * b + 1, pl.ds(0, SEQ_LEN - SPLIT)]],
                rows_v.at[b, pl.ds(SPLIT, SEQ_LEN - SPLIT)],
                sem,
            )
        for b in range(CB):
            pltpu.make_async_copy(
                tok_hbm.at[idx_v.at[2 * b]],
                rows_v.at[b, pl.ds(0, SPLIT)],
                sem,
            ).wait()
            pltpu.make_async_copy(
                tok_hbm.at[idx_v.at[2 * b + 1, pl.ds(0, SEQ_LEN - SPLIT)]],
                rows_v.at[b, pl.ds(SPLIT, SEQ_LEN - SPLIT)],
                sem,
            ).wait()

        def add_body(l, _):
            p0 = pos_v[l, 0:16]
            p1 = pos_v[l, 16:32]
            for b in range(CB):
                rows_v[b, l, 0:16] = rows_v[b, l, 0:16] + p0
                rows_v[b, l, 16:32] = rows_v[b, l, 16:32] + p1
            return 0

        lax.fori_loop(0, SEQ_LEN, add_body, 0)

        pltpu.sync_copy(
            rows_v, out_hbm.at[pl.ds(b0, CB), slice(None), pl.ds(0, EMBED)]
        )
        return 0

    lax.fori_loop(0, NCHUNKS, chunk_body, 0)


@jax.jit
def _run(tok, idx, pos):
    idx2 = jnp.pad(idx, ((0, 0), (0, SEQ_PAD - SEQ_LEN))).reshape(
        BATCH * 2, LROW
    )
    out = pl.kernel(
        _body,
        out_type=jax.ShapeDtypeStruct((BATCH, SEQ_LEN, PAD), jnp.float32),
        mesh=_MESH,
        scratch_types=[
            pltpu.VMEM((2 * CB, LROW), jnp.int32),
            pltpu.VMEM((CB, SEQ_LEN, EMBED), jnp.float32),
            pltpu.VMEM((SEQ_LEN, EMBED), jnp.float32),
            pltpu.SemaphoreType.DMA,
        ],
        compiler_params=pltpu.CompilerParams(use_tc_tiling_on_sc=False),
    )(tok, idx2, pos)
    return out[..., :EMBED]


def kernel(inputs, token_table, pos_table):
    return _run(token_table, inputs, pos_table)
